# baseline (speedup 1.0000x reference)
import functools

import numpy as np
import jax
import jax.numpy as jnp
from jax import lax
from jax.experimental import pallas as pl
from jax.experimental.pallas import tpu as pltpu
from jax.experimental.pallas import tpu_sc as plsc

N_TOK = 5
N_CLIENTS = 1000
D = 128
B = 1024
S = 200
MAIN = S - N_TOK
PREF_ROWS = N_CLIENTS * N_TOK

NC = 2
NS = 16
NW = NC * NS
B_PER_W = B // NW

_NOISE_JM = np.ascontiguousarray(
    np.asarray(
        jax.random.normal(jax.random.key(42), (N_CLIENTS, N_TOK, D),
                          dtype=jnp.float32)
    ).transpose(1, 0, 2)
)

_CBLK = 200


def _sample_body(a_ref, v_ref, n_ref, o_ref):
    for j in range(N_TOK):
        o_ref[j] = a_ref[:, j, :] + v_ref[:, j, :] * n_ref[j]


def _sample_table(avgs, vars_, noise_jm):
    nb = N_CLIENTS // _CBLK
    samp3d = pl.pallas_call(
        _sample_body,
        out_shape=jax.ShapeDtypeStruct((N_TOK, N_CLIENTS, D), jnp.float32),
        grid=(nb,),
        in_specs=[
            pl.BlockSpec((_CBLK, N_TOK, D), lambda i: (i, 0, 0)),
            pl.BlockSpec((_CBLK, N_TOK, D), lambda i: (i, 0, 0)),
            pl.BlockSpec((N_TOK, _CBLK, D), lambda i: (0, i, 0)),
        ],
        out_specs=pl.BlockSpec((N_TOK, _CBLK, D), lambda i: (0, i, 0)),
    )(avgs, vars_, noise_jm)
    return samp3d.reshape(PREF_ROWS, D)


def _sc_gather(tokens_flat, idx_pref, wte, samp):
    mesh = plsc.VectorSubcoreMesh(core_axis_name="c", subcore_axis_name="s")

    @functools.partial(
        pl.kernel,
        out_type=jax.ShapeDtypeStruct((B, S, D), jnp.float32),
        mesh=mesh,
        scratch_types=[
            pltpu.VMEM((B_PER_W * S,), jnp.int32),
            pltpu.VMEM((B_PER_W * 8,), jnp.int32),
            pltpu.VMEM((S + 8, D), jnp.float32),
            pltpu.VMEM((S + 8, D), jnp.float32),
            pltpu.VMEM((S + 8, D), jnp.float32),
            pltpu.VMEM((S + 8, D), jnp.float32),
            pltpu.SemaphoreType.DMA,
            pltpu.SemaphoreType.DMA,
            pltpu.SemaphoreType.DMA,
            pltpu.SemaphoreType.DMA,
            pltpu.SemaphoreType.DMA,
            pltpu.SemaphoreType.DMA,
            pltpu.SemaphoreType.DMA,
            pltpu.SemaphoreType.DMA,
        ],
    )
    def k(tok_hbm, idx_pref_hbm, wte_hbm, samp_hbm, out_hbm,
          idx_m_v, idx_p_v, blk0, blk1, blk2, blk3,
          sg0, sg1, sg2, sg3, sw0, sw1, sw2, sw3):
        wid = lax.axis_index("s") * NC + lax.axis_index("c")
        base = wid * B_PER_W
        blk = (blk0, blk1, blk2, blk3)
        sg = (sg0, sg1, sg2, sg3)
        sw = (sw0, sw1, sw2, sw3)

        pltpu.sync_copy(tok_hbm.at[pl.ds(base * S, B_PER_W * S)], idx_m_v)
        pltpu.sync_copy(idx_pref_hbm.at[pl.ds(base * 8, B_PER_W * 8)], idx_p_v)

        def start_gather(i, s):
            pltpu.async_copy(
                wte_hbm.at[idx_m_v.at[pl.ds(i * S, S)]],
                blk[s].at[pl.ds(0, S)], sg[s])
            pltpu.async_copy(
                samp_hbm.at[idx_p_v.at[pl.ds(i * 8, N_TOK)]],
                blk[s].at[pl.ds(S, N_TOK)], sg[s])

        def wait_gather(i, s):
            pltpu.make_async_copy(out_hbm.at[base + i],
                                  blk[s].at[pl.ds(0, S)], sg[s]).wait()
            pltpu.make_async_copy(samp_hbm.at[pl.ds(0, N_TOK)],
                                  blk[s].at[pl.ds(S, N_TOK)], sg[s]).wait()

        def start_write(i, s):
            pltpu.async_copy(blk[s].at[pl.ds(N_TOK, S)],
                             out_hbm.at[base + i], sw[s])

        def wait_write(i, s):
            pltpu.make_async_copy(blk[s].at[pl.ds(N_TOK, S)],
                                  out_hbm.at[base + i], sw[s]).wait()

        start_gather(0, 0)
        start_gather(1, 1)

        def group_body(g, _):
            for s in range(4):
                i = 4 * g + s
                sl2 = (s + 2) % 4

                @pl.when(i >= 2)
                def _():
                    wait_write(i - 2, sl2)

                @pl.when(i + 2 < B_PER_W)
                def _():
                    start_gather(i + 2, sl2)

                wait_gather(i, s)
                start_write(i, s)
            return ()

        lax.fori_loop(0, B_PER_W // 4, group_body, ())
        wait_write(B_PER_W - 2, 2)
        wait_write(B_PER_W - 1, 3)

    return k(tokens_flat, idx_pref, wte, samp)


@jax.jit
def kernel(tokens, wte_weight, avgs, vars_):
    samp = _sample_table(avgs, vars_, jnp.asarray(_NOISE_JM))

    cid = tokens[:, 0]
    pbase = (cid + N_CLIENTS - 1) % N_CLIENTS
    offs = jnp.array([0, 1000, 2000, 3000, 4000, 0, 0, 0], jnp.int32)
    idx_pref = pbase[:, None] + offs[None, :]

    return _sc_gather(tokens.reshape(-1), idx_pref.reshape(-1),
                      wte_weight, samp)


# speedup vs baseline: 1.0229x; 1.0229x over previous
import functools

import numpy as np
import jax
import jax.numpy as jnp
from jax import lax
from jax.experimental import pallas as pl
from jax.experimental.pallas import tpu as pltpu
from jax.experimental.pallas import tpu_sc as plsc

N_TOK = 5
N_CLIENTS = 1000
D = 128
B = 1024
S = 200
MAIN = S - N_TOK

NC = 2
NS = 16
NW = NC * NS
B_PER_W = B // NW

_NOISE = np.asarray(
    jax.random.normal(jax.random.key(42), (N_CLIENTS, N_TOK, D),
                      dtype=jnp.float32))


def _sc_gather(tokens_flat, idx_pref, wte, avgs, vars_, noise):
    mesh = plsc.VectorSubcoreMesh(core_axis_name="c", subcore_axis_name="s")

    @functools.partial(
        pl.kernel,
        out_type=jax.ShapeDtypeStruct((B, S, D), jnp.float32),
        mesh=mesh,
        scratch_types=[
            pltpu.VMEM((B_PER_W * S,), jnp.int32),
            pltpu.VMEM((B_PER_W * 8,), jnp.int32),
            pltpu.VMEM((S + 8, D), jnp.float32),
            pltpu.VMEM((S + 8, D), jnp.float32),
            pltpu.VMEM((S + 8, D), jnp.float32),
            pltpu.VMEM((S + 8, D), jnp.float32),
            pltpu.VMEM((1, N_TOK, D), jnp.float32),
            pltpu.VMEM((1, N_TOK, D), jnp.float32),
            pltpu.VMEM((1, N_TOK, D), jnp.float32),
            pltpu.VMEM((1, N_TOK, D), jnp.float32),
            pltpu.VMEM((1, N_TOK, D), jnp.float32),
            pltpu.VMEM((1, N_TOK, D), jnp.float32),
            pltpu.VMEM((1, N_TOK, D), jnp.float32),
            pltpu.VMEM((1, N_TOK, D), jnp.float32),
            pltpu.VMEM((1, N_TOK, D), jnp.float32),
            pltpu.VMEM((1, N_TOK, D), jnp.float32),
            pltpu.VMEM((1, N_TOK, D), jnp.float32),
            pltpu.VMEM((1, N_TOK, D), jnp.float32),
            pltpu.SemaphoreType.DMA,
            pltpu.SemaphoreType.DMA,
            pltpu.SemaphoreType.DMA,
            pltpu.SemaphoreType.DMA,
            pltpu.SemaphoreType.DMA,
            pltpu.SemaphoreType.DMA,
            pltpu.SemaphoreType.DMA,
            pltpu.SemaphoreType.DMA,
        ],
    )
    def k(tok_hbm, idx_pref_hbm, wte_hbm, avgs_hbm, vars_hbm, noise_hbm,
          out_hbm,
          idx_m_v, idx_p_v, blk0, blk1, blk2, blk3,
          ba0, ba1, ba2, ba3, bv0, bv1, bv2, bv3, bn0, bn1, bn2, bn3,
          sg0, sg1, sg2, sg3, sw0, sw1, sw2, sw3):
        wid = lax.axis_index("s") * NC + lax.axis_index("c")
        base = wid * B_PER_W
        blk = (blk0, blk1, blk2, blk3)
        ba = (ba0, ba1, ba2, ba3)
        bv = (bv0, bv1, bv2, bv3)
        bn = (bn0, bn1, bn2, bn3)
        sg = (sg0, sg1, sg2, sg3)
        sw = (sw0, sw1, sw2, sw3)

        pltpu.sync_copy(tok_hbm.at[pl.ds(base * S, B_PER_W * S)], idx_m_v)
        pltpu.sync_copy(idx_pref_hbm.at[pl.ds(base * 8, B_PER_W * 8)], idx_p_v)

        def start_gather(i, s):
            pidx = idx_p_v.at[pl.ds(i * 8, 1)]
            pltpu.async_copy(
                wte_hbm.at[idx_m_v.at[pl.ds(i * S, S)]],
                blk[s].at[pl.ds(0, S)], sg[s])
            pltpu.async_copy(avgs_hbm.at[pidx], ba[s], sg[s])
            pltpu.async_copy(vars_hbm.at[pidx], bv[s], sg[s])
            pltpu.async_copy(noise_hbm.at[pidx], bn[s], sg[s])

        def wait_gather(i, s):
            pltpu.make_async_copy(out_hbm.at[base + i],
                                  blk[s].at[pl.ds(0, S)], sg[s]).wait()
            pltpu.make_async_copy(avgs_hbm.at[pl.ds(0, 1)], ba[s], sg[s]).wait()
            pltpu.make_async_copy(avgs_hbm.at[pl.ds(0, 1)], bv[s], sg[s]).wait()
            pltpu.make_async_copy(avgs_hbm.at[pl.ds(0, 1)], bn[s], sg[s]).wait()

        def fma_prefix(s):
            for r in range(N_TOK):
                for c in range(D // 16):
                    sl = pl.ds(c * 16, 16)
                    blk[s][S + r, sl] = (
                        ba[s][0, r, sl] + bv[s][0, r, sl] * bn[s][0, r, sl])

        def start_write(i, s):
            pltpu.async_copy(blk[s].at[pl.ds(N_TOK, S)],
                             out_hbm.at[base + i], sw[s])

        def wait_write(i, s):
            pltpu.make_async_copy(blk[s].at[pl.ds(N_TOK, S)],
                                  out_hbm.at[base + i], sw[s]).wait()

        start_gather(0, 0)
        start_gather(1, 1)

        def group_body(g, _):
            for s in range(4):
                i = 4 * g + s
                sl2 = (s + 2) % 4

                @pl.when(i >= 2)
                def _():
                    wait_write(i - 2, sl2)

                @pl.when(i + 2 < B_PER_W)
                def _():
                    start_gather(i + 2, sl2)

                wait_gather(i, s)
                fma_prefix(s)
                start_write(i, s)
            return ()

        lax.fori_loop(0, B_PER_W // 4, group_body, ())
        wait_write(B_PER_W - 2, 2)
        wait_write(B_PER_W - 1, 3)

    return k(tokens_flat, idx_pref, wte, avgs, vars_, noise)


@jax.jit
def kernel(tokens, wte_weight, avgs, vars_):
    cid = tokens[:, 0]
    pbase = (cid + N_CLIENTS - 1) % N_CLIENTS
    idx_pref = jnp.zeros((B, 8), jnp.int32).at[:, 0].set(pbase)

    return _sc_gather(tokens.reshape(-1), idx_pref.reshape(-1),
                      wte_weight, avgs, vars_, jnp.asarray(_NOISE))


# speedup vs baseline: 1.0564x; 1.0327x over previous
import functools

import numpy as np
import jax
import jax.numpy as jnp
from jax import lax
from jax.experimental import pallas as pl
from jax.experimental.pallas import tpu as pltpu
from jax.experimental.pallas import tpu_sc as plsc

N_TOK = 5
N_CLIENTS = 1000
D = 128
B = 1024
S = 200
MAIN = S - N_TOK
PREF_ROWS = N_CLIENTS * N_TOK

NC = 2
NS = 16
NW = NC * NS
B_PER_W = B // NW

_NOISE = np.asarray(
    jax.random.normal(jax.random.key(42), (N_CLIENTS, N_TOK, D),
                      dtype=jnp.float32))


def _sc_gather(tokens_flat, idx_pref, wte, samp):
    mesh = plsc.VectorSubcoreMesh(core_axis_name="c", subcore_axis_name="s")

    @functools.partial(
        pl.kernel,
        out_type=jax.ShapeDtypeStruct((B, S, D), jnp.float32),
        mesh=mesh,
        scratch_types=[
            pltpu.VMEM((B_PER_W * S,), jnp.int32),
            pltpu.VMEM((B_PER_W * 8,), jnp.int32),
            pltpu.VMEM((S + 8, D), jnp.float32),
            pltpu.VMEM((S + 8, D), jnp.float32),
            pltpu.VMEM((S + 8, D), jnp.float32),
            pltpu.VMEM((S + 8, D), jnp.float32),
            pltpu.SemaphoreType.DMA,
            pltpu.SemaphoreType.DMA,
            pltpu.SemaphoreType.DMA,
            pltpu.SemaphoreType.DMA,
            pltpu.SemaphoreType.DMA,
            pltpu.SemaphoreType.DMA,
            pltpu.SemaphoreType.DMA,
            pltpu.SemaphoreType.DMA,
        ],
    )
    def k(tok_hbm, idx_pref_hbm, wte_hbm, samp_hbm, out_hbm,
          idx_m_v, idx_p_v, blk0, blk1, blk2, blk3,
          sg0, sg1, sg2, sg3, sw0, sw1, sw2, sw3):
        wid = lax.axis_index("s") * NC + lax.axis_index("c")
        base = wid * B_PER_W
        blk = (blk0, blk1, blk2, blk3)
        sg = (sg0, sg1, sg2, sg3)
        sw = (sw0, sw1, sw2, sw3)

        pltpu.sync_copy(tok_hbm.at[pl.ds(base * S, B_PER_W * S)], idx_m_v)
        pltpu.sync_copy(idx_pref_hbm.at[pl.ds(base * 8, B_PER_W * 8)], idx_p_v)

        def start_gather(i, s):
            pltpu.async_copy(
                wte_hbm.at[idx_m_v.at[pl.ds(i * S, S)]],
                blk[s].at[pl.ds(0, S)], sg[s])
            pltpu.async_copy(
                samp_hbm.at[idx_p_v.at[pl.ds(i * 8, N_TOK)]],
                blk[s].at[pl.ds(S, N_TOK)], sg[s])

        def wait_gather(i, s):
            pltpu.make_async_copy(out_hbm.at[base + i],
                                  blk[s].at[pl.ds(0, S)], sg[s]).wait()
            pltpu.make_async_copy(samp_hbm.at[pl.ds(0, N_TOK)],
                                  blk[s].at[pl.ds(S, N_TOK)], sg[s]).wait()

        def start_write(i, s):
            pltpu.async_copy(blk[s].at[pl.ds(N_TOK, S)],
                             out_hbm.at[base + i], sw[s])

        def wait_write(i, s):
            pltpu.make_async_copy(blk[s].at[pl.ds(N_TOK, S)],
                                  out_hbm.at[base + i], sw[s]).wait()

        start_gather(0, 0)
        start_gather(1, 1)

        def group_body(g, _):
            for s in range(4):
                i = 4 * g + s
                sl2 = (s + 2) % 4

                @pl.when(i >= 2)
                def _():
                    wait_write(i - 2, sl2)

                @pl.when(i + 2 < B_PER_W)
                def _():
                    start_gather(i + 2, sl2)

                wait_gather(i, s)
                start_write(i, s)
            return ()

        lax.fori_loop(0, B_PER_W // 4, group_body, ())
        wait_write(B_PER_W - 2, 2)
        wait_write(B_PER_W - 1, 3)

    return k(tokens_flat, idx_pref, wte, samp)


@jax.jit
def kernel(tokens, wte_weight, avgs, vars_):
    samp = (avgs + vars_ * jnp.asarray(_NOISE)).reshape(PREF_ROWS, D)

    cid = tokens[:, 0]
    pbase = ((cid + N_CLIENTS - 1) % N_CLIENTS) * N_TOK
    offs = jnp.array([0, 1, 2, 3, 4, 0, 0, 0], jnp.int32)
    idx_pref = pbase[:, None] + offs[None, :]

    return _sc_gather(tokens.reshape(-1), idx_pref.reshape(-1),
                      wte_weight, samp)


# speedup vs baseline: 1.0730x; 1.0158x over previous
import functools

import numpy as np
import jax
import jax.numpy as jnp
from jax import lax
from jax.experimental import pallas as pl
from jax.experimental.pallas import tpu as pltpu
from jax.experimental.pallas import tpu_sc as plsc

N_TOK = 5
N_CLIENTS = 1000
D = 128
B = 1024
S = 200
MAIN = S - N_TOK
PREF_ROWS = N_CLIENTS * N_TOK

NC = 2
NS = 16
NW = NC * NS
B_PER_W = B // NW

_NOISE = np.asarray(
    jax.random.normal(jax.random.key(42), (N_CLIENTS, N_TOK, D),
                      dtype=jnp.float32))


def _sc_gather(tokens_flat, idx_pref, wte, samp):
    mesh = plsc.VectorSubcoreMesh(core_axis_name="c", subcore_axis_name="s")

    @functools.partial(
        pl.kernel,
        out_type=jax.ShapeDtypeStruct((B, S, D), jnp.float32),
        mesh=mesh,
        scratch_types=[
            pltpu.VMEM((B_PER_W * S,), jnp.int32),
            pltpu.VMEM((B_PER_W * 8,), jnp.int32),
            pltpu.VMEM((S + 8, D), jnp.float32),
            pltpu.VMEM((S + 8, D), jnp.float32),
            pltpu.SemaphoreType.DMA,
            pltpu.SemaphoreType.DMA,
            pltpu.SemaphoreType.DMA,
            pltpu.SemaphoreType.DMA,
        ],
    )
    def k(tok_hbm, idx_pref_hbm, wte_hbm, samp_hbm, out_hbm,
          idx_m_v, idx_p_v, blk0, blk1,
          sg0, sg1, sw0, sw1):
        wid = lax.axis_index("s") * NC + lax.axis_index("c")
        base = wid * B_PER_W
        blk = (blk0, blk1)
        sg = (sg0, sg1)
        sw = (sw0, sw1)

        pltpu.sync_copy(tok_hbm.at[pl.ds(base * S, B_PER_W * S)], idx_m_v)
        pltpu.sync_copy(idx_pref_hbm.at[pl.ds(base * 8, B_PER_W * 8)], idx_p_v)

        def start_gather(i, s):
            pltpu.async_copy(
                wte_hbm.at[idx_m_v.at[pl.ds(i * S, S)]],
                blk[s].at[pl.ds(0, S)], sg[s])
            pltpu.async_copy(
                samp_hbm.at[idx_p_v.at[pl.ds(i * 8, N_TOK)]],
                blk[s].at[pl.ds(S, N_TOK)], sg[s])

        def wait_gather(i, s):
            pltpu.make_async_copy(out_hbm.at[base + i],
                                  blk[s].at[pl.ds(0, S)], sg[s]).wait()
            pltpu.make_async_copy(samp_hbm.at[pl.ds(0, N_TOK)],
                                  blk[s].at[pl.ds(S, N_TOK)], sg[s]).wait()

        def start_write(i, s):
            pltpu.async_copy(blk[s].at[pl.ds(N_TOK, S)],
                             out_hbm.at[base + i], sw[s])

        def wait_write(i, s):
            pltpu.make_async_copy(blk[s].at[pl.ds(N_TOK, S)],
                                  out_hbm.at[base + i], sw[s]).wait()

        start_gather(0, 0)
        npair = B_PER_W // 2

        def pair_body(p, _):
            i0 = 2 * p
            i1 = i0 + 1

            @pl.when(p >= 1)
            def _():
                wait_write(i0 - 1, 1)

            start_gather(i1, 1)
            wait_gather(i0, 0)
            start_write(i0, 0)

            @pl.when(p + 1 < npair)
            def _():
                wait_write(i0, 0)
                start_gather(i0 + 2, 0)

            wait_gather(i1, 1)
            start_write(i1, 1)
            return ()

        lax.fori_loop(0, npair, pair_body, ())
        wait_write(B_PER_W - 2, 0)
        wait_write(B_PER_W - 1, 1)

    return k(tokens_flat, idx_pref, wte, samp)


@jax.jit
def kernel(tokens, wte_weight, avgs, vars_):
    samp = (avgs + vars_ * jnp.asarray(_NOISE)).reshape(PREF_ROWS, D)

    cid = tokens[:, 0]
    pbase = ((cid + N_CLIENTS - 1) % N_CLIENTS) * N_TOK
    offs = jnp.array([0, 1, 2, 3, 4, 0, 0, 0], jnp.int32)
    idx_pref = pbase[:, None] + offs[None, :]

    return _sc_gather(tokens.reshape(-1), idx_pref.reshape(-1),
                      wte_weight, samp)
